# Initial kernel scaffold; baseline (speedup 1.0000x reference)
import jax
import jax.numpy as jnp
from jax.experimental import pallas as pl


def kernel(search_xyz, search_feature, estimation_cla, template_box, W1, bn_gamma, bn_beta, bn_mean, bn_var, W2, b2):
    raise NotImplementedError("write your pallas kernel here")


# speedup vs baseline: 4.6855x; 4.6855x over previous
import jax
import jax.numpy as jnp
from jax.experimental import pallas as pl

_NUM_PRE = 900
_NUM_OUT = 64
_THRESH = 0.85
_EPS = 1e-5
_BLK = 2048


def _reg_head_body(xyzT_ref, feat_ref, claT_ref, W1_ref, scale_ref, shift_ref,
                   W2_ref, b2_ref, prop_ref, score_ref):
    x = feat_ref[0]
    h = jax.lax.dot_general(W1_ref[...], x, (((1,), (0,)), ((), ())),
                            precision=jax.lax.Precision.HIGHEST,
                            preferred_element_type=jnp.float32)
    h = h * scale_ref[...] + shift_ref[...]
    h = jnp.maximum(h, 0.0)
    off = jax.lax.dot_general(W2_ref[...], h, (((1,), (0,)), ((), ())),
                              precision=jax.lax.Precision.HIGHEST,
                              preferred_element_type=jnp.float32)
    off = off + b2_ref[...]
    vote = xyzT_ref[0] + off[0:3, :]
    prop_ref[0] = jnp.concatenate([vote, off[3:4, :]], axis=0)
    score_ref[0] = jax.nn.sigmoid(claT_ref[0])


def _nms_body(p4T_ref, boxes_ref, scores_ref, bb_ref, ss_ref):
    p = p4T_ref[0]
    cx = p[0:1, :]
    cz = p[2:3, :]
    w = boxes_ref[0, 0, 4]
    l = boxes_ref[0, 0, 5]
    x1 = cx - w / 2.0
    x2 = cx + w / 2.0
    z1 = cz - l / 2.0
    z2 = cz + l / 2.0
    area = (x2 - x1) * (z2 - z1)
    iota = jax.lax.broadcasted_iota(jnp.int32, (1, _NUM_PRE), 1)
    jrow = jax.lax.broadcasted_iota(jnp.int32, (_NUM_OUT, 1), 0)

    def cond(c):
        i, cnt, _, _ = c
        return jnp.logical_and(i < _NUM_PRE, cnt < _NUM_OUT)

    def body(c):
        i, cnt, sup, mt = c
        oh = iota == i
        x1i = jnp.sum(jnp.where(oh, x1, 0.0), keepdims=True)
        x2i = jnp.sum(jnp.where(oh, x2, 0.0), keepdims=True)
        z1i = jnp.sum(jnp.where(oh, z1, 0.0), keepdims=True)
        z2i = jnp.sum(jnp.where(oh, z2, 0.0), keepdims=True)
        ai = jnp.sum(jnp.where(oh, area, 0.0), keepdims=True)
        supi = jnp.sum(jnp.where(oh, sup, 0.0))
        alive = supi <= 0.5
        ix1 = jnp.maximum(x1i, x1)
        ix2 = jnp.minimum(x2i, x2)
        iz1 = jnp.maximum(z1i, z1)
        iz2 = jnp.minimum(z2i, z2)
        inter = jnp.maximum(ix2 - ix1, 0.0) * jnp.maximum(iz2 - iz1, 0.0)
        union = ai + area - inter
        iou = inter / jnp.maximum(union, 1e-8)
        hit = jnp.logical_and(iou > _THRESH, iota > i)
        cand = jnp.maximum(sup, jnp.where(hit, 1.0, 0.0))
        sup = jnp.where(alive, cand, sup)
        mt = jnp.where(jnp.logical_and(jnp.logical_and(jrow == cnt, oh), alive),
                       1.0, mt)
        cnt = cnt + jnp.where(alive, 1, 0)
        return (i + 1, cnt, sup, mt)

    init = (jnp.int32(0), jnp.int32(0),
            jnp.zeros((1, _NUM_PRE), jnp.float32),
            jnp.zeros((_NUM_OUT, _NUM_PRE), jnp.float32))
    _, _, _, mt = jax.lax.while_loop(cond, body, init)

    bb_ref[0] = jax.lax.dot_general(mt, boxes_ref[0], (((1,), (0,)), ((), ())),
                                    precision=jax.lax.Precision.HIGHEST,
                                    preferred_element_type=jnp.float32)
    ss_ref[0] = jax.lax.dot_general(mt, scores_ref[0], (((1,), (0,)), ((), ())),
                                    precision=jax.lax.Precision.HIGHEST,
                                    preferred_element_type=jnp.float32)


def kernel(search_xyz, search_feature, estimation_cla, template_box,
           W1, bn_gamma, bn_beta, bn_mean, bn_var, W2, b2):
    B, C, N = search_feature.shape
    nb = (N + _BLK - 1) // _BLK

    scale = (bn_gamma / jnp.sqrt(bn_var + _EPS))[:, None]
    shift = (bn_beta - bn_mean * (bn_gamma / jnp.sqrt(bn_var + _EPS)))[:, None]
    xyzT = jnp.transpose(search_xyz, (0, 2, 1))
    claT = jnp.transpose(estimation_cla, (0, 2, 1))

    prop4T, scores = pl.pallas_call(
        _reg_head_body,
        grid=(B, nb),
        in_specs=[
            pl.BlockSpec((1, 3, _BLK), lambda b, n: (b, 0, n)),
            pl.BlockSpec((1, C, _BLK), lambda b, n: (b, 0, n)),
            pl.BlockSpec((1, 1, _BLK), lambda b, n: (b, 0, n)),
            pl.BlockSpec((128, C), lambda b, n: (0, 0)),
            pl.BlockSpec((128, 1), lambda b, n: (0, 0)),
            pl.BlockSpec((128, 1), lambda b, n: (0, 0)),
            pl.BlockSpec((4, 128), lambda b, n: (0, 0)),
            pl.BlockSpec((4, 1), lambda b, n: (0, 0)),
        ],
        out_specs=[
            pl.BlockSpec((1, 4, _BLK), lambda b, n: (b, 0, n)),
            pl.BlockSpec((1, 1, _BLK), lambda b, n: (b, 0, n)),
        ],
        out_shape=[
            jax.ShapeDtypeStruct((B, 4, N), jnp.float32),
            jax.ShapeDtypeStruct((B, 1, N), jnp.float32),
        ],
    )(xyzT, search_feature, claT, W1, scale, shift, W2, b2[:, None])

    prop4 = jnp.transpose(prop4T, (0, 2, 1))

    top_s, top_i = jax.lax.top_k(scores[:, 0, :], _NUM_PRE)
    p4o = jnp.take_along_axis(prop4, top_i[:, :, None], axis=1)
    dims = jnp.broadcast_to(template_box[:, :, 3:6], (B, _NUM_PRE, 3))
    boxes8 = jnp.concatenate(
        [p4o[:, :, 0:3], dims, p4o[:, :, 3:4],
         jnp.zeros((B, _NUM_PRE, 1), jnp.float32)], axis=2)
    p4oT = jnp.transpose(p4o, (0, 2, 1))

    bb8, ss = pl.pallas_call(
        _nms_body,
        grid=(B,),
        in_specs=[
            pl.BlockSpec((1, 4, _NUM_PRE), lambda b: (b, 0, 0)),
            pl.BlockSpec((1, _NUM_PRE, 8), lambda b: (b, 0, 0)),
            pl.BlockSpec((1, _NUM_PRE, 1), lambda b: (b, 0, 0)),
        ],
        out_specs=[
            pl.BlockSpec((1, _NUM_OUT, 8), lambda b: (b, 0, 0)),
            pl.BlockSpec((1, _NUM_OUT, 1), lambda b: (b, 0, 0)),
        ],
        out_shape=[
            jax.ShapeDtypeStruct((B, _NUM_OUT, 8), jnp.float32),
            jax.ShapeDtypeStruct((B, _NUM_OUT, 1), jnp.float32),
        ],
    )(p4oT, boxes8, top_s[:, :, None])

    ret_bbox3d = bb8[:, :, 0:7]
    ret_scores = ss[:, :, 0]
    center_xyzs = ret_bbox3d[:, :, 0:3]
    return (ret_bbox3d, ret_scores, prop4, center_xyzs)
